# speedup vs baseline: 1.8312x; 1.8312x over previous
import math

import jax
import jax.numpy as jnp
from jax.experimental import pallas as pl
from jax.experimental.pallas import tpu as pltpu

_PAD = 0
_SMOOTHING = 0.1
_CONF = 1.0 - _SMOOTHING

_W = 2048


def _make_body(size, n_blocks, smooth, c1):
    def _body(t_ref, x_ref, out_ref, s_ref, g_ref, x0_ref):
        c = pl.program_id(0)
        xblk = x_ref[...]
        t = t_ref[...]
        cols = c * _W + jax.lax.broadcasted_iota(jnp.int32, (1, _W), 1)
        xv = jnp.where(cols < size, xblk, 0.0)
        eq = cols == t
        rpart = jnp.sum(xv, axis=1, keepdims=True)
        gpart = jnp.sum(jnp.where(eq, xv, 0.0), axis=1, keepdims=True)

        @pl.when(c == 0)
        def _():
            s_ref[...] = rpart
            g_ref[...] = gpart
            x0_ref[...] = xblk[:, 0:1]

        @pl.when(c > 0)
        def _():
            s_ref[...] += rpart
            g_ref[...] += gpart

        @pl.when(c == n_blocks - 1)
        def _():
            mask = (t != _PAD).astype(jnp.float32)
            contrib = (
                c1
                - smooth * (s_ref[...] - x0_ref[...] - g_ref[...])
                - _CONF * g_ref[...]
            )
            out_ref[0, 0] = jnp.sum(mask * contrib)

    return _body


def kernel(x, target):
    b, size = x.shape
    n_blocks = (size + _W - 1) // _W
    smooth = _SMOOTHING / (size - 2)
    c1 = (size - 2) * smooth * math.log(smooth) + _CONF * math.log(_CONF)
    t2 = target.astype(jnp.int32).reshape(b, 1)
    out = pl.pallas_call(
        _make_body(size, n_blocks, smooth, c1),
        grid=(n_blocks,),
        in_specs=[
            pl.BlockSpec((b, 1), lambda c: (0, 0)),
            pl.BlockSpec((b, _W), lambda c: (0, c)),
        ],
        out_specs=pl.BlockSpec((1, 1), lambda c: (0, 0), memory_space=pltpu.SMEM),
        out_shape=jax.ShapeDtypeStruct((1, 1), jnp.float32),
        scratch_shapes=[
            pltpu.VMEM((b, 1), jnp.float32),
            pltpu.VMEM((b, 1), jnp.float32),
            pltpu.VMEM((b, 1), jnp.float32),
        ],
        compiler_params=pltpu.CompilerParams(
            dimension_semantics=("arbitrary",),
        ),
    )(t2, x)
    return out[0, 0]
